# baseline (speedup 1.0000x reference)
import functools

import jax
import jax.numpy as jnp
from jax import lax
from jax.experimental import pallas as pl
from jax.experimental.pallas import tpu as pltpu
from jax.experimental.pallas import tpu_sc as plsc

BINS = 2048
ROWS = 8192
COLS = 4096
_info = plsc.get_sparse_core_info()
NC, NS, L = _info.num_cores, _info.num_subcores, _info.num_lanes
NW = NC * NS
RPW = ROWS // NW
RPC = 8
NCH = RPW // RPC
CB = COLS // L
U = 8

_mesh = plsc.VectorSubcoreMesh(core_axis_name="c", subcore_axis_name="s")
_params = pltpu.CompilerParams(
    needs_layout_passes=False, use_tc_tiling_on_sc=True)

_gdn = lax.GatherDimensionNumbers(
    offset_dims=(), collapsed_slice_dims=(0,), start_index_map=(0,))


def _permute(v, idx):
    return lax.gather(
        v, idx[:, None], _gdn, slice_sizes=(1,),
        unique_indices=True, indices_are_sorted=False,
        mode=lax.GatherScatterMode.PROMISE_IN_BOUNDS)


@functools.partial(
    pl.kernel,
    mesh=_mesh,
    out_type=[
        jax.ShapeDtypeStruct((NW * L,), jnp.float32),
        jax.ShapeDtypeStruct((NW * L,), jnp.float32),
    ],
    scratch_types=[
        pltpu.VMEM((2, RPC, COLS), jnp.float32),
        pltpu.VMEM((L,), jnp.float32),
        pltpu.VMEM((L,), jnp.float32),
        pltpu.SemaphoreType.DMA,
        pltpu.SemaphoreType.DMA,
    ],
    compiler_params=_params,
)
def _minmax_k(x_hbm, min_hbm, max_hbm, xbuf, mn_v, mx_v, sem0, sem1):
    wid = lax.axis_index("s") * NC + lax.axis_index("c")
    row0 = wid * RPW
    sems = (sem0, sem1)

    def _start(chunk, b):
        pltpu.make_async_copy(
            x_hbm.at[pl.ds(row0 + chunk * RPC, RPC)], xbuf.at[b], sems[b]
        ).start()

    def _wait(b):
        pltpu.make_async_copy(
            x_hbm.at[pl.ds(0, RPC)], xbuf.at[b], sems[b]
        ).wait()

    _start(0, 0)

    def outer_body(c2, carry):
        mn, mx = carry
        for b in range(2):
            chunk = c2 * 2 + b

            @pl.when(chunk + 1 < NCH)
            def _():
                _start(chunk + 1, 1 - b)

            _wait(b)

            def vec_body(i, carry2):
                mns, mxs = carry2
                vs = [xbuf[b, j, pl.ds(i * L, L)] for j in range(U)]
                mns = tuple(jnp.minimum(m, v) for m, v in zip(mns, vs))
                mxs = tuple(jnp.maximum(m, v) for m, v in zip(mxs, vs))
                return mns, mxs

            mn, mx = lax.fori_loop(0, CB, vec_body, (mn, mx))
        return mn, mx

    inf = jnp.full((L,), jnp.inf, jnp.float32)
    ninf = jnp.full((L,), -jnp.inf, jnp.float32)
    init = ((inf,) * U, (ninf,) * U)
    mns, mxs = lax.fori_loop(0, NCH // 2, outer_body, init)
    mn = mns[0]
    mx = mxs[0]
    for j in range(1, U):
        mn = jnp.minimum(mn, mns[j])
        mx = jnp.maximum(mx, mxs[j])
    mn_v[...] = mn
    mx_v[...] = mx
    pltpu.sync_copy(mn_v, min_hbm.at[pl.ds(wid * L, L)])
    pltpu.sync_copy(mx_v, max_hbm.at[pl.ds(wid * L, L)])


@functools.partial(
    pl.kernel,
    mesh=_mesh,
    out_type=jax.ShapeDtypeStruct((NW, BINS), jnp.float32),
    scratch_types=[
        pltpu.VMEM((2, RPC, COLS), jnp.float32),
        pltpu.VMEM((NW * L,), jnp.float32),
        pltpu.VMEM((NW * L,), jnp.float32),
        pltpu.VMEM((BINS * L,), jnp.float32),
        pltpu.VMEM((BINS,), jnp.float32),
        pltpu.SemaphoreType.DMA,
        pltpu.SemaphoreType.DMA,
    ],
    compiler_params=_params,
)
def _hist_k(x_hbm, pmin_hbm, pmax_hbm, out_hbm, xbuf, pmin_v, pmax_v,
            hist_v, hout_v, sem0, sem1):
    wid = lax.axis_index("s") * NC + lax.axis_index("c")
    row0 = wid * RPW

    pltpu.sync_copy(pmin_hbm, pmin_v)
    pltpu.sync_copy(pmax_hbm, pmax_v)

    def red_body(i, carry):
        mn, mx = carry
        return (jnp.minimum(mn, pmin_v[pl.ds(i * L, L)]),
                jnp.maximum(mx, pmax_v[pl.ds(i * L, L)]))

    mnv, mxv = lax.fori_loop(
        0, NW, red_body,
        (jnp.full((L,), jnp.inf, jnp.float32),
         jnp.full((L,), -jnp.inf, jnp.float32)))
    lane = lax.iota(jnp.int32, L)
    perms = [lane ^ s for s in (8, 4, 2, 1)]
    for p in perms:
        mnv = jnp.minimum(mnv, _permute(mnv, p))
        mxv = jnp.maximum(mxv, _permute(mxv, p))
    mns = mnv[0]
    mxs = mxv[0]

    rng = mxs - mns
    rmin = mns - 0.5 * rng
    rmax = mxs + 0.5 * rng
    bw = (rmax - rmin) * (1.0 / BINS)
    rmin2 = rmin + bw * (1.0 / 32.0)
    invv = 1.0 / jnp.full((L,), bw, jnp.float32)
    cv = jnp.full((L,), 2.0 ** 19, jnp.float32) - jnp.full(
        (L,), rmin2, jnp.float32) * invv

    zero16 = jnp.zeros((L,), jnp.float32)

    def z_body(b, _):
        hist_v[pl.ds(b * L, L)] = zero16
        return 0

    lax.fori_loop(0, BINS, z_body, 0)

    one16 = jnp.full((L,), 1.0, jnp.float32)
    sems = (sem0, sem1)

    def _start(chunk, b):
        pltpu.make_async_copy(
            x_hbm.at[pl.ds(row0 + chunk * RPC, RPC)], xbuf.at[b], sems[b]
        ).start()

    def _wait(b):
        pltpu.make_async_copy(
            x_hbm.at[pl.ds(0, RPC)], xbuf.at[b], sems[b]
        ).wait()

    _start(0, 0)

    def outer_body(c2, _):
        for b in range(2):
            chunk = c2 * 2 + b

            @pl.when(chunk + 1 < NCH)
            def _():
                _start(chunk + 1, 1 - b)

            _wait(b)

            @plsc.parallel_loop(0, CB)
            def vec_body(i):
                vs = [xbuf[b, j, pl.ds(i * L, L)] for j in range(U)]
                ts = [v * invv + cv for v in vs]
                bs = [lax.bitcast_convert_type(t, jnp.int32) for t in ts]
                addrs = [(bj & 0x7FF0) | lane for bj in bs]
                for a in addrs:
                    plsc.addupdate_scatter(hist_v, [a], one16)
        return 0

    lax.fori_loop(0, NCH // 2, outer_body, 0)

    def f_body(g, _):
        out = jnp.zeros((L,), jnp.float32)
        for j in range(L):
            v = hist_v[pl.ds((g * L + j) * L, L)]
            for p in perms:
                v = v + _permute(v, p)
            out = jnp.where(lane == j, v, out)
        hout_v[pl.ds(g * L, L)] = out
        return 0

    lax.fori_loop(0, BINS // L, f_body, 0)
    pltpu.sync_copy(hout_v, out_hbm.at[wid])


def kernel(x):
    mn, mx = _minmax_k(x)
    parts = _hist_k(x, mn, mx)
    return jnp.sum(parts, axis=0)


# speedup vs baseline: 1.0101x; 1.0101x over previous
import functools

import jax
import jax.numpy as jnp
from jax import lax
from jax.experimental import pallas as pl
from jax.experimental.pallas import tpu as pltpu
from jax.experimental.pallas import tpu_sc as plsc

BINS = 2048
ROWS = 8192
COLS = 4096
_info = plsc.get_sparse_core_info()
NC, NS, L = _info.num_cores, _info.num_subcores, _info.num_lanes
NW = NC * NS
RPW = ROWS // NW
RPC = 8
NCH = RPW // RPC
CB = COLS // L
U = 8

_mesh = plsc.VectorSubcoreMesh(core_axis_name="c", subcore_axis_name="s")
_params = pltpu.CompilerParams(
    needs_layout_passes=False, use_tc_tiling_on_sc=True)

_gdn = lax.GatherDimensionNumbers(
    offset_dims=(), collapsed_slice_dims=(0,), start_index_map=(0,))


def _permute(v, idx):
    return lax.gather(
        v, idx[:, None], _gdn, slice_sizes=(1,),
        unique_indices=True, indices_are_sorted=False,
        mode=lax.GatherScatterMode.PROMISE_IN_BOUNDS)


@functools.partial(
    pl.kernel,
    mesh=_mesh,
    out_type=[
        jax.ShapeDtypeStruct((NW * L,), jnp.float32),
        jax.ShapeDtypeStruct((NW * L,), jnp.float32),
    ],
    scratch_types=[
        pltpu.VMEM((2, RPC, COLS), jnp.float32),
        pltpu.VMEM((L,), jnp.float32),
        pltpu.VMEM((L,), jnp.float32),
        pltpu.SemaphoreType.DMA,
        pltpu.SemaphoreType.DMA,
    ],
    compiler_params=_params,
)
def _minmax_k(x_hbm, min_hbm, max_hbm, xbuf, mn_v, mx_v, sem0, sem1):
    wid = lax.axis_index("s") * NC + lax.axis_index("c")
    row0 = wid * RPW
    sems = (sem0, sem1)

    def _start(chunk, b):
        pltpu.make_async_copy(
            x_hbm.at[pl.ds(row0 + chunk * RPC, RPC)], xbuf.at[b], sems[b]
        ).start()

    def _wait(b):
        pltpu.make_async_copy(
            x_hbm.at[pl.ds(0, RPC)], xbuf.at[b], sems[b]
        ).wait()

    _start(0, 0)

    def outer_body(c2, carry):
        mn, mx = carry
        for b in range(2):
            chunk = c2 * 2 + b

            @pl.when(chunk + 1 < NCH)
            def _():
                _start(chunk + 1, 1 - b)

            _wait(b)

            def vec_body(i, carry2):
                mns, mxs = carry2
                vs = [xbuf[b, j, pl.ds(i * L, L)] for j in range(U)]
                mns = tuple(jnp.minimum(m, v) for m, v in zip(mns, vs))
                mxs = tuple(jnp.maximum(m, v) for m, v in zip(mxs, vs))
                return mns, mxs

            mn, mx = lax.fori_loop(0, CB, vec_body, (mn, mx))
        return mn, mx

    inf = jnp.full((L,), jnp.inf, jnp.float32)
    ninf = jnp.full((L,), -jnp.inf, jnp.float32)
    init = ((inf,) * U, (ninf,) * U)
    mns, mxs = lax.fori_loop(0, NCH // 2, outer_body, init)
    mn = mns[0]
    mx = mxs[0]
    for j in range(1, U):
        mn = jnp.minimum(mn, mns[j])
        mx = jnp.maximum(mx, mxs[j])
    mn_v[...] = mn
    mx_v[...] = mx
    pltpu.sync_copy(mn_v, min_hbm.at[pl.ds(wid * L, L)])
    pltpu.sync_copy(mx_v, max_hbm.at[pl.ds(wid * L, L)])


@functools.partial(
    pl.kernel,
    mesh=_mesh,
    out_type=jax.ShapeDtypeStruct((NW, BINS), jnp.float32),
    scratch_types=[
        pltpu.VMEM((2, RPC, COLS), jnp.float32),
        pltpu.VMEM((NW * L,), jnp.float32),
        pltpu.VMEM((NW * L,), jnp.float32),
        pltpu.VMEM((BINS * L,), jnp.float32),
        pltpu.VMEM((BINS,), jnp.float32),
        pltpu.SemaphoreType.DMA,
        pltpu.SemaphoreType.DMA,
    ],
    compiler_params=_params,
)
def _hist_k(x_hbm, pmin_hbm, pmax_hbm, out_hbm, xbuf, pmin_v, pmax_v,
            hist_v, hout_v, sem0, sem1):
    wid = lax.axis_index("s") * NC + lax.axis_index("c")
    row0 = wid * RPW

    pltpu.sync_copy(pmin_hbm, pmin_v)
    pltpu.sync_copy(pmax_hbm, pmax_v)

    def red_body(i, carry):
        mn, mx = carry
        return (jnp.minimum(mn, pmin_v[pl.ds(i * L, L)]),
                jnp.maximum(mx, pmax_v[pl.ds(i * L, L)]))

    mnv, mxv = lax.fori_loop(
        0, NW, red_body,
        (jnp.full((L,), jnp.inf, jnp.float32),
         jnp.full((L,), -jnp.inf, jnp.float32)))
    lane = lax.iota(jnp.int32, L)
    perms = [lane ^ s for s in (8, 4, 2, 1)]
    for p in perms:
        mnv = jnp.minimum(mnv, _permute(mnv, p))
        mxv = jnp.maximum(mxv, _permute(mxv, p))
    mns = mnv[0]
    mxs = mxv[0]

    rng = mxs - mns
    rmin = mns - 0.5 * rng
    rmax = mxs + 0.5 * rng
    bw = (rmax - rmin) * (1.0 / BINS)
    rmin2 = rmin + bw * (1.0 / 32.0)
    rminv = jnp.full((L,), rmin2, jnp.float32)
    invv = 1.0 / jnp.full((L,), bw, jnp.float32)
    magicv = jnp.full((L,), 2.0 ** 19, jnp.float32)

    zero16 = jnp.zeros((L,), jnp.float32)

    def z_body(b, _):
        hist_v[pl.ds(b * L, L)] = zero16
        return 0

    lax.fori_loop(0, BINS, z_body, 0)

    one16 = jnp.full((L,), 1.0, jnp.float32)
    sems = (sem0, sem1)

    def _start(chunk, b):
        pltpu.make_async_copy(
            x_hbm.at[pl.ds(row0 + chunk * RPC, RPC)], xbuf.at[b], sems[b]
        ).start()

    def _wait(b):
        pltpu.make_async_copy(
            x_hbm.at[pl.ds(0, RPC)], xbuf.at[b], sems[b]
        ).wait()

    _start(0, 0)

    def outer_body(c2, _):
        for b in range(2):
            chunk = c2 * 2 + b

            @pl.when(chunk + 1 < NCH)
            def _():
                _start(chunk + 1, 1 - b)

            _wait(b)

            @plsc.parallel_loop(0, CB, unroll=2)
            def vec_body(i):
                vs = [xbuf[b, j, pl.ds(i * L, L)] for j in range(U)]
                ts = [(v - rminv) * invv + magicv for v in vs]
                bs = [lax.bitcast_convert_type(t, jnp.int32) for t in ts]
                addrs = [(bj & 0x7FF0) | lane for bj in bs]
                for a in addrs:
                    plsc.addupdate_scatter(hist_v, [a], one16)
        return 0

    lax.fori_loop(0, NCH // 2, outer_body, 0)

    def f_body(g, _):
        out = jnp.zeros((L,), jnp.float32)
        for j in range(L):
            v = hist_v[pl.ds((g * L + j) * L, L)]
            for p in perms:
                v = v + _permute(v, p)
            out = jnp.where(lane == j, v, out)
        hout_v[pl.ds(g * L, L)] = out
        return 0

    lax.fori_loop(0, BINS // L, f_body, 0)
    pltpu.sync_copy(hout_v, out_hbm.at[wid])


def kernel(x):
    mn, mx = _minmax_k(x)
    parts = _hist_k(x, mn, mx)
    return jnp.sum(parts, axis=0)


# speedup vs baseline: 1.0327x; 1.0224x over previous
import functools

import jax
import jax.numpy as jnp
from jax import lax
from jax.experimental import pallas as pl
from jax.experimental.pallas import tpu as pltpu
from jax.experimental.pallas import tpu_sc as plsc

BINS = 2048
ROWS = 8192
COLS = 4096
_info = plsc.get_sparse_core_info()
NC, NS, L = _info.num_cores, _info.num_subcores, _info.num_lanes
NW = NC * NS
RPW = ROWS // NW
RPC = 8
NCH = RPW // RPC
CB = COLS // L
U = 8

_mesh = plsc.VectorSubcoreMesh(core_axis_name="c", subcore_axis_name="s")
_params = pltpu.CompilerParams(
    needs_layout_passes=False, use_tc_tiling_on_sc=True)

_gdn = lax.GatherDimensionNumbers(
    offset_dims=(), collapsed_slice_dims=(0,), start_index_map=(0,))


def _permute(v, idx):
    return lax.gather(
        v, idx[:, None], _gdn, slice_sizes=(1,),
        unique_indices=True, indices_are_sorted=False,
        mode=lax.GatherScatterMode.PROMISE_IN_BOUNDS)


@functools.partial(
    pl.kernel,
    mesh=_mesh,
    out_type=[
        jax.ShapeDtypeStruct((NW * L,), jnp.float32),
        jax.ShapeDtypeStruct((NW * L,), jnp.float32),
    ],
    scratch_types=[
        pltpu.VMEM((2, RPC, COLS), jnp.float32),
        pltpu.VMEM((L,), jnp.float32),
        pltpu.VMEM((L,), jnp.float32),
        pltpu.SemaphoreType.DMA,
        pltpu.SemaphoreType.DMA,
    ],
    compiler_params=_params,
)
def _minmax_k(x_hbm, min_hbm, max_hbm, xbuf, mn_v, mx_v, sem0, sem1):
    wid = lax.axis_index("s") * NC + lax.axis_index("c")
    row0 = wid * RPW
    sems = (sem0, sem1)

    def _start(chunk, b):
        pltpu.make_async_copy(
            x_hbm.at[pl.ds(row0 + chunk * RPC, RPC)], xbuf.at[b], sems[b]
        ).start()

    def _wait(b):
        pltpu.make_async_copy(
            x_hbm.at[pl.ds(0, RPC)], xbuf.at[b], sems[b]
        ).wait()

    _start(0, 0)

    def outer_body(c2, carry):
        mn, mx = carry
        for b in range(2):
            chunk = c2 * 2 + b

            @pl.when(chunk + 1 < NCH)
            def _():
                _start(chunk + 1, 1 - b)

            _wait(b)

            def vec_body(i, carry2):
                mns, mxs = carry2
                vs = [xbuf[b, j, pl.ds(i * L, L)] for j in range(U)]
                mns = tuple(jnp.minimum(m, v) for m, v in zip(mns, vs))
                mxs = tuple(jnp.maximum(m, v) for m, v in zip(mxs, vs))
                return mns, mxs

            mn, mx = lax.fori_loop(0, CB, vec_body, (mn, mx))
        return mn, mx

    inf = jnp.full((L,), jnp.inf, jnp.float32)
    ninf = jnp.full((L,), -jnp.inf, jnp.float32)
    init = ((inf,) * U, (ninf,) * U)
    mns, mxs = lax.fori_loop(0, NCH // 2, outer_body, init)
    mn = mns[0]
    mx = mxs[0]
    for j in range(1, U):
        mn = jnp.minimum(mn, mns[j])
        mx = jnp.maximum(mx, mxs[j])
    mn_v[...] = mn
    mx_v[...] = mx
    pltpu.sync_copy(mn_v, min_hbm.at[pl.ds(wid * L, L)])
    pltpu.sync_copy(mx_v, max_hbm.at[pl.ds(wid * L, L)])


def _mm_body(x_ref, mn_ref, mx_ref):
    i = pl.program_id(0)

    @pl.when(i == 0)
    def _():
        mn_ref[...] = jnp.full((8, 128), jnp.inf, jnp.float32)
        mx_ref[...] = jnp.full((8, 128), -jnp.inf, jnp.float32)

    xb = x_ref[...]
    mn_ref[...] = jnp.minimum(mn_ref[...], jnp.min(xb))
    mx_ref[...] = jnp.maximum(mx_ref[...], jnp.max(xb))


_minmax_tc = pl.pallas_call(
    _mm_body,
    grid=(ROWS // RPW,),
    in_specs=[pl.BlockSpec((RPW, COLS), lambda i: (i, 0))],
    out_specs=[
        pl.BlockSpec((8, 128), lambda i: (0, 0)),
        pl.BlockSpec((8, 128), lambda i: (0, 0)),
    ],
    out_shape=[
        jax.ShapeDtypeStruct((8, 128), jnp.float32),
        jax.ShapeDtypeStruct((8, 128), jnp.float32),
    ],
)


@functools.partial(
    pl.kernel,
    mesh=_mesh,
    out_type=jax.ShapeDtypeStruct((NW, BINS), jnp.float32),
    scratch_types=[
        pltpu.VMEM((2, RPC, COLS), jnp.float32),
        pltpu.VMEM((8, 128), jnp.float32),
        pltpu.VMEM((8, 128), jnp.float32),
        pltpu.VMEM((BINS * L,), jnp.float32),
        pltpu.VMEM((BINS,), jnp.float32),
        pltpu.SemaphoreType.DMA,
        pltpu.SemaphoreType.DMA,
    ],
    compiler_params=_params,
)
def _hist_k(x_hbm, pmin_hbm, pmax_hbm, out_hbm, xbuf, pmin_v, pmax_v,
            hist_v, hout_v, sem0, sem1):
    wid = lax.axis_index("s") * NC + lax.axis_index("c")
    row0 = wid * RPW

    pltpu.sync_copy(pmin_hbm, pmin_v)
    pltpu.sync_copy(pmax_hbm, pmax_v)

    mnv = pmin_v[0, pl.ds(0, L)]
    mxv = pmax_v[0, pl.ds(0, L)]
    for r in range(8):
        for q in range(128 // L):
            if r == 0 and q == 0:
                continue
            mnv = jnp.minimum(mnv, pmin_v[r, pl.ds(q * L, L)])
            mxv = jnp.maximum(mxv, pmax_v[r, pl.ds(q * L, L)])
    lane = lax.iota(jnp.int32, L)
    perms = [lane ^ s for s in (8, 4, 2, 1)]
    for p in perms:
        mnv = jnp.minimum(mnv, _permute(mnv, p))
        mxv = jnp.maximum(mxv, _permute(mxv, p))
    mns = mnv[0]
    mxs = mxv[0]

    rng = mxs - mns
    rmin = mns - 0.5 * rng
    rmax = mxs + 0.5 * rng
    bw = (rmax - rmin) * (1.0 / BINS)
    rmin2 = rmin + bw * (1.0 / 32.0)
    rminv = jnp.full((L,), rmin2, jnp.float32)
    invv = 1.0 / jnp.full((L,), bw, jnp.float32)
    magicv = jnp.full((L,), 2.0 ** 19, jnp.float32)

    zero16 = jnp.zeros((L,), jnp.float32)

    def z_body(b, _):
        hist_v[pl.ds(b * L, L)] = zero16
        return 0

    lax.fori_loop(0, BINS, z_body, 0)

    one16 = jnp.full((L,), 1.0, jnp.float32)
    sems = (sem0, sem1)

    def _start(chunk, b):
        pltpu.make_async_copy(
            x_hbm.at[pl.ds(row0 + chunk * RPC, RPC)], xbuf.at[b], sems[b]
        ).start()

    def _wait(b):
        pltpu.make_async_copy(
            x_hbm.at[pl.ds(0, RPC)], xbuf.at[b], sems[b]
        ).wait()

    _start(0, 0)

    def outer_body(c2, _):
        for b in range(2):
            chunk = c2 * 2 + b

            @pl.when(chunk + 1 < NCH)
            def _():
                _start(chunk + 1, 1 - b)

            _wait(b)

            @plsc.parallel_loop(0, CB)
            def vec_body(i):
                vs = [xbuf[b, j, pl.ds(i * L, L)] for j in range(U)]
                ts = [(v - rminv) * invv + magicv for v in vs]
                bs = [lax.bitcast_convert_type(t, jnp.int32) for t in ts]
                addrs = [(bj & 0x7FF0) | lane for bj in bs]
                for a in addrs:
                    plsc.addupdate_scatter(hist_v, [a], one16)
        return 0

    lax.fori_loop(0, NCH // 2, outer_body, 0)

    def f_body(g, _):
        out = jnp.zeros((L,), jnp.float32)
        for j in range(L):
            v = hist_v[pl.ds((g * L + j) * L, L)]
            for p in perms:
                v = v + _permute(v, p)
            out = jnp.where(lane == j, v, out)
        hout_v[pl.ds(g * L, L)] = out
        return 0

    lax.fori_loop(0, BINS // L, f_body, 0)
    pltpu.sync_copy(hout_v, out_hbm.at[wid])


def kernel(x):
    mn, mx = _minmax_tc(x)
    parts = _hist_k(x, mn, mx)
    return jnp.sum(parts, axis=0)
